# baseline (speedup 1.0000x reference)
import jax
import jax.numpy as jnp
from jax.experimental import pallas as pl
from jax.experimental.pallas import tpu as pltpu

B = 1024
N_CTX = 4
D = 512
ROWS = 77
P_PRE, P_MID, P_SUF = 5, 2, 62
OFF_CLS = P_PRE
OFF_MID = OFF_CLS + N_CTX
OFF_CLO = OFF_MID + P_MID
OFF_SUF = OFF_CLO + N_CTX

PPB = 7
NBLK = ROWS // PPB
_ORDER = [3, 4, 5, 6, 7, 8, 9, 10, 0, 1, 2]
_FIRST_GATHER_STEP = _ORDER.index(0)

GI_STEPS = _FIRST_GATHER_STEP
EPG = B // GI_STEPS
N_CLOTH = 1000


def _asm_body(lbl_s, clo_s, ord_s, cls_hbm, clo_hbm, tmpl_ref, out_ref,
              cls_st, clo_st, g_sem):
    i = pl.program_id(0)

    @pl.when(i < GI_STEPS)
    def _issue():
        for e in range(EPG):
            b = i * EPG + e
            pltpu.make_async_copy(cls_hbm.at[lbl_s[b]], cls_st.at[b],
                                  g_sem.at[0]).start()
            pltpu.make_async_copy(clo_hbm.at[clo_s[b]], clo_st.at[b],
                                  g_sem.at[1]).start()

    @pl.when(i == _FIRST_GATHER_STEP)
    def _drain():
        pltpu.make_async_copy(cls_hbm.at[pl.ds(0, B)], cls_st,
                              g_sem.at[0]).wait()
        pltpu.make_async_copy(clo_hbm.at[pl.ds(0, N_CLOTH)],
                              clo_st.at[pl.ds(0, N_CLOTH)],
                              g_sem.at[1]).wait()
        pltpu.make_async_copy(clo_hbm.at[pl.ds(0, B - N_CLOTH)],
                              clo_st.at[pl.ds(N_CLOTH, B - N_CLOTH)],
                              g_sem.at[1]).wait()

    for j, blk in enumerate(_ORDER):
        @pl.when(i == j)
        def _emit(blk=blk):
            dmas = []
            for q in range(PPB):
                p = PPB * blk + q
                if OFF_CLS <= p < OFF_MID:
                    d = pltpu.make_async_copy(cls_st.at[:, p - OFF_CLS],
                                              out_ref.at[q], g_sem.at[0])
                elif OFF_CLO <= p < OFF_SUF:
                    d = pltpu.make_async_copy(clo_st.at[:, p - OFF_CLO],
                                              out_ref.at[q], g_sem.at[1])
                else:
                    out_ref[q] = jnp.broadcast_to(tmpl_ref[p], (B, D))
                    continue
                d.start()
                dmas.append(d)
            for d in dmas:
                d.wait()


@jax.jit
def _prompt_assemble(label, cloth_label, order, cls_ctx, clo_ctx, tmpl_full):
    grid_spec = pltpu.PrefetchScalarGridSpec(
        num_scalar_prefetch=3,
        grid=(NBLK,),
        in_specs=[
            pl.BlockSpec(memory_space=pltpu.MemorySpace.HBM),
            pl.BlockSpec(memory_space=pltpu.MemorySpace.HBM),
            pl.BlockSpec((ROWS, D), lambda i, lbl, clo, o: (0, 0)),
        ],
        out_specs=pl.BlockSpec((PPB, B, D),
                               lambda i, lbl, clo, o: (o[i], 0, 0)),
        scratch_shapes=[
            pltpu.VMEM((B, N_CTX, D), jnp.float32),
            pltpu.VMEM((B, N_CTX, D), jnp.float32),
            pltpu.SemaphoreType.DMA((2,)),
        ],
    )
    return pl.pallas_call(
        _asm_body,
        grid_spec=grid_spec,
        out_shape=jax.ShapeDtypeStruct((ROWS, B, D), jnp.float32),
        compiler_params=pltpu.CompilerParams(
            dimension_semantics=("arbitrary",),
            vmem_limit_bytes=50 * 1024 * 1024),
    )(label, cloth_label, order, cls_ctx, clo_ctx, tmpl_full)


def kernel(label, cloth_label, cls_ctx, cls_cloth_ctx,
           token_prefix, token_middle, token_suffix):
    zeros4 = jnp.zeros((N_CTX, D), jnp.float32)
    tmpl_full = jnp.concatenate(
        [token_prefix.reshape(P_PRE, D), zeros4,
         token_middle.reshape(P_MID, D), zeros4,
         token_suffix.reshape(P_SUF, D)], axis=0)
    order = jnp.asarray(_ORDER, dtype=jnp.int32)
    out77 = _prompt_assemble(label.astype(jnp.int32),
                             cloth_label.astype(jnp.int32),
                             order, cls_ctx, cls_cloth_ctx, tmpl_full)
    return (jnp.transpose(out77, (1, 0, 2)), 17)


# speedup vs baseline: 1.0819x; 1.0819x over previous
import jax
import jax.numpy as jnp
from jax.experimental import pallas as pl
from jax.experimental.pallas import tpu as pltpu

B = 1024
N_CTX = 4
D = 512
ROWS = 77
P_PRE, P_MID, P_SUF = 5, 2, 62
OFF_CLS = P_PRE
OFF_MID = OFF_CLS + N_CTX
OFF_CLO = OFF_MID + P_MID
OFF_SUF = OFF_CLO + N_CTX

PPB = 7
NBLK = ROWS // PPB
_ORDER = [3, 4, 5, 6, 7, 8, 9, 10, 0, 1, 2]
_FIRST_GATHER_STEP = _ORDER.index(0)

GI_STEPS = _FIRST_GATHER_STEP
EPG = B // GI_STEPS
N_CLOTH = 1000


def _asm_body(lbl_s, clo_s, ord_s, cls_hbm, clo_hbm, tmpl_ref, out_ref,
              cls_st, clo_st, g_sem):
    i = pl.program_id(0)

    @pl.when(i < GI_STEPS)
    def _issue():
        for e in range(EPG):
            b = i * EPG + e
            pltpu.make_async_copy(cls_hbm.at[lbl_s[b]], cls_st.at[b],
                                  g_sem.at[0]).start()
            pltpu.make_async_copy(clo_hbm.at[clo_s[b]], clo_st.at[b],
                                  g_sem.at[1]).start()

    @pl.when(i == _FIRST_GATHER_STEP)
    def _drain():
        pltpu.make_async_copy(cls_hbm.at[pl.ds(0, B)], cls_st,
                              g_sem.at[0]).wait()
        pltpu.make_async_copy(clo_hbm.at[pl.ds(0, N_CLOTH)],
                              clo_st.at[pl.ds(0, N_CLOTH)],
                              g_sem.at[1]).wait()
        pltpu.make_async_copy(clo_hbm.at[pl.ds(0, B - N_CLOTH)],
                              clo_st.at[pl.ds(N_CLOTH, B - N_CLOTH)],
                              g_sem.at[1]).wait()

    for j, blk in enumerate(_ORDER):
        @pl.when(i == j)
        def _emit(blk=blk):
            for q in range(PPB):
                p = PPB * blk + q
                if OFF_CLS <= p < OFF_MID:
                    out_ref[q] = cls_st[:, p - OFF_CLS, :]
                elif OFF_CLO <= p < OFF_SUF:
                    out_ref[q] = clo_st[:, p - OFF_CLO, :]
                else:
                    out_ref[q] = jnp.broadcast_to(tmpl_ref[p], (B, D))


@jax.jit
def _prompt_assemble(label, cloth_label, order, cls_ctx, clo_ctx, tmpl_full):
    grid_spec = pltpu.PrefetchScalarGridSpec(
        num_scalar_prefetch=3,
        grid=(NBLK,),
        in_specs=[
            pl.BlockSpec(memory_space=pltpu.MemorySpace.HBM),
            pl.BlockSpec(memory_space=pltpu.MemorySpace.HBM),
            pl.BlockSpec((ROWS, D), lambda i, lbl, clo, o: (0, 0)),
        ],
        out_specs=pl.BlockSpec((PPB, B, D),
                               lambda i, lbl, clo, o: (o[i], 0, 0)),
        scratch_shapes=[
            pltpu.VMEM((B, N_CTX, D), jnp.float32),
            pltpu.VMEM((B, N_CTX, D), jnp.float32),
            pltpu.SemaphoreType.DMA((2,)),
        ],
    )
    return pl.pallas_call(
        _asm_body,
        grid_spec=grid_spec,
        out_shape=jax.ShapeDtypeStruct((ROWS, B, D), jnp.float32),
        compiler_params=pltpu.CompilerParams(
            dimension_semantics=("arbitrary",),
            vmem_limit_bytes=50 * 1024 * 1024),
    )(label, cloth_label, order, cls_ctx, clo_ctx, tmpl_full)


def kernel(label, cloth_label, cls_ctx, cls_cloth_ctx,
           token_prefix, token_middle, token_suffix):
    zeros4 = jnp.zeros((N_CTX, D), jnp.float32)
    tmpl_full = jnp.concatenate(
        [token_prefix.reshape(P_PRE, D), zeros4,
         token_middle.reshape(P_MID, D), zeros4,
         token_suffix.reshape(P_SUF, D)], axis=0)
    order = jnp.asarray(_ORDER, dtype=jnp.int32)
    out77 = _prompt_assemble(label.astype(jnp.int32),
                             cloth_label.astype(jnp.int32),
                             order, cls_ctx, cls_cloth_ctx, tmpl_full)
    return (jnp.transpose(out77, (1, 0, 2)), 17)
